# baseline (speedup 1.0000x reference)
import functools

import jax
import jax.numpy as jnp
from jax import lax
from jax.experimental import pallas as pl
from jax.experimental.pallas import tpu as pltpu
from jax.experimental.pallas import tpu_sc as plsc

DIMS = [119, 4, 12, 12, 10, 6, 6, 2, 2]
OFFS = [0]
for _d in DIMS[:-1]:
    OFFS.append(OFFS[-1] + _d)
TROWS = sum(DIMS)
NF = len(DIMS)
D = 64
L = 16
NC, NS = 2, 16
NW = NC * NS
N = 100000
BLK = 112
NFULL = N // BLK
TAIL = N - NFULL * BLK
NBLOCKS = NFULL + 1
BPW = (NBLOCKS + NW - 1) // NW


def _sc_body(x_hbm, t_hbm, out_hbm, xbuf, idxbuf, gbuf, obuf, sem):
    wid = lax.axis_index("s") * NC + lax.axis_index("c")
    lanes = lax.iota(jnp.int32, L)

    def do_block(base, S):
        pltpu.sync_copy(x_hbm.at[pl.ds(base * NF, S * NF)], xbuf.at[pl.ds(0, S * NF)])
        for v in range(S // L):
            rows = (lanes + (L * v)) * NF
            for i in range(NF):
                xi = plsc.load_gather(xbuf, [rows + i])
                idxbuf[i, pl.ds(L * v, L)] = xi + OFFS[i]
        copies = []
        for i in range(NF):
            copies.append(
                pltpu.async_copy(
                    t_hbm.at[idxbuf.at[i, pl.ds(0, S)]],
                    gbuf.at[i, pl.ds(0, S)],
                    sem,
                )
            )
        for c in copies:
            c.wait()

        def acc_row(r, carry):
            for cb in range(D // L):
                s = gbuf[0, r, pl.ds(cb * L, L)]
                for i in range(1, NF):
                    s = s + gbuf[i, r, pl.ds(cb * L, L)]
                obuf[r, pl.ds(cb * L, L)] = s
            return carry

        lax.fori_loop(0, S, acc_row, 0)
        pltpu.sync_copy(obuf.at[pl.ds(0, S)], out_hbm.at[pl.ds(base, S)])

    def block_step(j, carry):
        b = wid + j * NW

        @pl.when(b < NFULL)
        def _():
            do_block(b * BLK, BLK)

        @pl.when(b == NFULL)
        def _():
            do_block(NFULL * BLK, TAIL)

        return carry

    lax.fori_loop(0, BPW, block_step, 0)


@jax.jit
def kernel(x, W0, W1, W2, W3, W4, W5, W6, W7, W8):
    x = x.astype(jnp.int32).reshape(N * NF)
    table = jnp.concatenate([W0, W1, W2, W3, W4, W5, W6, W7, W8], axis=0)
    mesh = plsc.VectorSubcoreMesh(core_axis_name="c", subcore_axis_name="s")
    run = pl.kernel(
        _sc_body,
        out_type=jax.ShapeDtypeStruct((N, D), jnp.float32),
        mesh=mesh,
        scratch_types=[
            pltpu.VMEM((BLK * NF,), jnp.int32),
            pltpu.VMEM((NF, BLK), jnp.int32),
            pltpu.VMEM((NF, BLK, D), jnp.float32),
            pltpu.VMEM((BLK, D), jnp.float32),
            pltpu.SemaphoreType.DMA,
        ],
        compiler_params=pltpu.CompilerParams(
            needs_layout_passes=False, use_tc_tiling_on_sc=False
        ),
    )
    return run(x, table)


# speedup vs baseline: 2.7110x; 2.7110x over previous
import functools

import jax
import jax.numpy as jnp
from jax import lax
from jax.experimental import pallas as pl
from jax.experimental.pallas import tpu as pltpu
from jax.experimental.pallas import tpu_sc as plsc

NF = 9
D = 64
L = 16
NC, NS = 2, 16
NW = NC * NS
N = 100000
BLK = 112
NFULL = N // BLK
TAIL = N - NFULL * BLK
NBLOCKS = NFULL + 1
BPW = (NBLOCKS + NW - 1) // NW

OFF0 = 0
OFF1 = 119
OFF2 = 135
OFF3 = 279
TROWS = 639

WOFF = {1: 0, 2: 4, 3: 16, 4: 28, 5: 38, 6: 44, 7: 50, 8: 52}
WROWS = 54


def _sc_body(x_hbm, w_hbm, w0_hbm, out_hbm, xbuf, tbuf, wbuf, t56, obuf):
    wid = lax.axis_index("s") * NC + lax.axis_index("c")
    lanes = lax.iota(jnp.int32, L)

    pltpu.sync_copy(w0_hbm, tbuf.at[pl.ds(0, 119 * D)])
    pltpu.sync_copy(w_hbm, wbuf)

    def w_row(i, r):
        return (WOFF[i] + r) * D

    for r in range(16):
        a, b, c = r // 4, (r // 2) % 2, r % 2
        for cb in range(D // L):
            v = (
                wbuf[pl.ds(w_row(1, a) + cb * L, L)]
                + wbuf[pl.ds(w_row(7, b) + cb * L, L)]
                + wbuf[pl.ds(w_row(8, c) + cb * L, L)]
            )
            tbuf[pl.ds((OFF1 + r) * D + cb * L, L)] = v

    def g2_row(r, carry):
        a = r // 12
        b = r - a * 12
        for cb in range(D // L):
            v = wbuf[pl.ds(w_row(2, a) + cb * L, L)] + wbuf[pl.ds(w_row(3, b) + cb * L, L)]
            tbuf[pl.ds((OFF2 + r) * D + cb * L, L)] = v
        return carry

    lax.fori_loop(0, 144, g2_row, 0)

    def t56_row(r, carry):
        a = r // 6
        b = r - a * 6
        for cb in range(D // L):
            v = wbuf[pl.ds(w_row(5, a) + cb * L, L)] + wbuf[pl.ds(w_row(6, b) + cb * L, L)]
            t56[pl.ds(r * D + cb * L, L)] = v
        return carry

    lax.fori_loop(0, 36, t56_row, 0)

    def g3_row(r, carry):
        a = r // 36
        s = r - a * 36
        for cb in range(D // L):
            v = wbuf[pl.ds(w_row(4, a) + cb * L, L)] + t56[pl.ds(s * D + cb * L, L)]
            tbuf[pl.ds((OFF3 + r) * D + cb * L, L)] = v
        return carry

    lax.fori_loop(0, 360, g3_row, 0)

    def do_block(base, S):
        pltpu.sync_copy(x_hbm.at[pl.ds(base * NF, S * NF)], xbuf.at[pl.ds(0, S * NF)])

        def sub(v, carry):
            av = lanes + L * v
            rowsx = av * NF
            xs = [plsc.load_gather(xbuf, [rowsx + i]) for i in range(NF)]
            b0 = (xs[0] + OFF0) * D
            b1 = (((xs[1] * 2 + xs[7]) * 2 + xs[8]) + OFF1) * D
            b2 = ((xs[2] * 12 + xs[3]) + OFF2) * D
            b3 = ((((xs[4] * 6 + xs[5]) * 6 + xs[6])) + OFF3) * D
            sb = av * D
            for c in range(D):
                acc = plsc.load_gather(tbuf, [b0 + c])
                acc = acc + plsc.load_gather(tbuf, [b1 + c])
                acc = acc + plsc.load_gather(tbuf, [b2 + c])
                acc = acc + plsc.load_gather(tbuf, [b3 + c])
                plsc.store_scatter(obuf, [sb + c], acc)
            return carry

        lax.fori_loop(0, S // L, sub, 0)
        pltpu.sync_copy(obuf.at[pl.ds(0, S * D)], out_hbm.at[pl.ds(base * D, S * D)])

    def block_step(j, carry):
        b = wid + j * NW

        @pl.when(b < NFULL)
        def _():
            do_block(b * BLK, BLK)

        @pl.when(b == NFULL)
        def _():
            do_block(NFULL * BLK, TAIL)

        return carry

    lax.fori_loop(0, BPW, block_step, 0)


@jax.jit
def kernel(x, W0, W1, W2, W3, W4, W5, W6, W7, W8):
    x = x.astype(jnp.int32).reshape(N * NF)
    wcat = jnp.concatenate(
        [W1, W2, W3, W4, W5, W6, W7, W8], axis=0
    ).reshape(WROWS * D)
    w0 = W0.reshape(119 * D)
    mesh = plsc.VectorSubcoreMesh(core_axis_name="c", subcore_axis_name="s")
    run = pl.kernel(
        _sc_body,
        out_type=jax.ShapeDtypeStruct((N * D,), jnp.float32),
        mesh=mesh,
        scratch_types=[
            pltpu.VMEM((BLK * NF,), jnp.int32),
            pltpu.VMEM((TROWS * D,), jnp.float32),
            pltpu.VMEM((WROWS * D,), jnp.float32),
            pltpu.VMEM((36 * D,), jnp.float32),
            pltpu.VMEM((BLK * D,), jnp.float32),
        ],
        compiler_params=pltpu.CompilerParams(
            needs_layout_passes=False, use_tc_tiling_on_sc=False
        ),
    )
    return run(x, wcat, w0).reshape(N, D)


# speedup vs baseline: 3.0527x; 1.1261x over previous
import functools

import jax
import jax.numpy as jnp
from jax import lax
from jax.experimental import pallas as pl
from jax.experimental.pallas import tpu as pltpu
from jax.experimental.pallas import tpu_sc as plsc

NF = 9
D = 64
L = 16
NC, NS = 2, 16
NW = NC * NS
N = 100000
APW = 3200
APW_LAST = N - 31 * APW
CHUNK = 400
CW = CHUNK * D
NSUB = CHUNK // L

OFF0 = 0
OFF1 = 119
OFF2 = 135
OFF3 = 279
TROWS = 639

WOFF = {1: 0, 2: 4, 3: 16, 4: 28, 5: 38, 6: 44, 7: 50, 8: 52}
WROWS = 54


def _sc_body(x_hbm, w_hbm, w0_hbm, out_hbm, xbuf, tbuf, wbuf, obuf, sem, xsem):
    wid = lax.axis_index("s") * NC + lax.axis_index("c")
    lanes = lax.iota(jnp.int32, L)
    start = wid * APW

    tcopy0 = pltpu.async_copy(w0_hbm, tbuf.at[pl.ds(0, 119 * D)], sem)
    tcopy1 = pltpu.async_copy(w_hbm, wbuf, sem)

    @pl.when(wid < 31)
    def _():
        pltpu.async_copy(
            x_hbm.at[pl.ds(start * NF, APW * NF)], xbuf.at[pl.ds(0, APW * NF)], xsem
        )

    @pl.when(wid == 31)
    def _():
        pltpu.async_copy(
            x_hbm.at[pl.ds(31 * APW * NF, APW_LAST * NF)],
            xbuf.at[pl.ds(0, APW_LAST * NF)],
            xsem,
        )

    tcopy0.wait()
    tcopy1.wait()

    def w_row(i, r):
        return (WOFF[i] + r) * D

    for r in range(16):
        a, b, c = r // 4, (r // 2) % 2, r % 2
        for cb in range(D // L):
            v = (
                wbuf[pl.ds(w_row(1, a) + cb * L, L)]
                + wbuf[pl.ds(w_row(7, b) + cb * L, L)]
                + wbuf[pl.ds(w_row(8, c) + cb * L, L)]
            )
            tbuf[pl.ds((OFF1 + r) * D + cb * L, L)] = v

    def g2_row(r, carry):
        a = r // 12
        b = r - a * 12
        for cb in range(D // L):
            v = wbuf[pl.ds(w_row(2, a) + cb * L, L)] + wbuf[pl.ds(w_row(3, b) + cb * L, L)]
            tbuf[pl.ds((OFF2 + r) * D + cb * L, L)] = v
        return carry

    lax.fori_loop(0, 144, g2_row, 0)

    def g3_row(r, carry):
        a = r // 36
        s = r - a * 36
        b = s // 6
        c = s - b * 6
        for cb in range(D // L):
            v = (
                wbuf[pl.ds(w_row(4, a) + cb * L, L)]
                + wbuf[pl.ds(w_row(5, b) + cb * L, L)]
                + wbuf[pl.ds(w_row(6, c) + cb * L, L)]
            )
            tbuf[pl.ds((OFF3 + r) * D + cb * L, L)] = v
        return carry

    lax.fori_loop(0, 360, g3_row, 0)

    @pl.when(wid < 31)
    def _():
        pltpu.make_async_copy(
            x_hbm.at[pl.ds(0, APW * NF)], xbuf.at[pl.ds(0, APW * NF)], xsem
        ).wait()

    @pl.when(wid == 31)
    def _():
        pltpu.make_async_copy(
            x_hbm.at[pl.ds(0, APW_LAST * NF)], xbuf.at[pl.ds(0, APW_LAST * NF)], xsem
        ).wait()

    nchunks = jnp.where(wid == 31, APW_LAST // CHUNK, APW // CHUNK)

    def chunk(j, carry):
        p = j & 1
        pbase = p * CW
        cb_atoms = j * CHUNK

        @pl.when(j >= 2)
        def _():
            pltpu.make_async_copy(
                obuf.at[pl.ds(0, CW)], out_hbm.at[pl.ds(0, CW)], sem
            ).wait()

        def sub(v, carry2):
            av = cb_atoms + lanes + L * v
            rowsx = av * NF
            xs = [plsc.load_gather(xbuf, [rowsx + i]) for i in range(NF)]
            b0 = (xs[0] + OFF0) * D
            b1 = (((xs[1] * 2 + xs[7]) * 2 + xs[8]) + OFF1) * D
            b2 = ((xs[2] * 12 + xs[3]) + OFF2) * D
            b3 = ((((xs[4] * 6 + xs[5]) * 6 + xs[6])) + OFF3) * D
            sb = pbase + (lanes + L * v) * D
            for c in range(0, D, 2):
                a0 = plsc.load_gather(tbuf, [b0 + c])
                a1 = plsc.load_gather(tbuf, [b1 + c])
                a2 = plsc.load_gather(tbuf, [b2 + c])
                a3 = plsc.load_gather(tbuf, [b3 + c])
                e0 = plsc.load_gather(tbuf, [b0 + (c + 1)])
                e1 = plsc.load_gather(tbuf, [b1 + (c + 1)])
                e2 = plsc.load_gather(tbuf, [b2 + (c + 1)])
                e3 = plsc.load_gather(tbuf, [b3 + (c + 1)])
                s = (a0 + a1) + (a2 + a3)
                t = (e0 + e1) + (e2 + e3)
                plsc.store_scatter(obuf, [sb + c], s)
                plsc.store_scatter(obuf, [sb + (c + 1)], t)
            return carry2

        lax.fori_loop(0, NSUB, sub, 0)

        pltpu.async_copy(
            obuf.at[pl.ds(pbase, CW)],
            out_hbm.at[pl.ds((start + cb_atoms) * D, CW)],
            sem,
        )
        return carry

    lax.fori_loop(0, nchunks, chunk, 0)

    pltpu.make_async_copy(
        obuf.at[pl.ds(0, CW)], out_hbm.at[pl.ds(0, CW)], sem
    ).wait()
    pltpu.make_async_copy(
        obuf.at[pl.ds(0, CW)], out_hbm.at[pl.ds(0, CW)], sem
    ).wait()


@jax.jit
def kernel(x, W0, W1, W2, W3, W4, W5, W6, W7, W8):
    x = x.astype(jnp.int32).reshape(N * NF)
    wcat = jnp.concatenate(
        [W1, W2, W3, W4, W5, W6, W7, W8], axis=0
    ).reshape(WROWS * D)
    w0 = W0.reshape(119 * D)
    mesh = plsc.VectorSubcoreMesh(core_axis_name="c", subcore_axis_name="s")
    run = pl.kernel(
        _sc_body,
        out_type=jax.ShapeDtypeStruct((N * D,), jnp.float32),
        mesh=mesh,
        scratch_types=[
            pltpu.VMEM((APW * NF,), jnp.int32),
            pltpu.VMEM((TROWS * D,), jnp.float32),
            pltpu.VMEM((WROWS * D,), jnp.float32),
            pltpu.VMEM((2 * CW,), jnp.float32),
            pltpu.SemaphoreType.DMA,
            pltpu.SemaphoreType.DMA,
        ],
        compiler_params=pltpu.CompilerParams(
            needs_layout_passes=False, use_tc_tiling_on_sc=False
        ),
    )
    return run(x, wcat, w0).reshape(N, D)
